# Initial kernel scaffold; baseline (speedup 1.0000x reference)
import functools

import jax
import jax.numpy as jnp
from jax import lax
from jax.experimental import pallas as pl
from jax.experimental.pallas import tpu as pltpu
from jax.experimental.pallas import tpu_sc as plsc

N_NODES = 10000
D = 128
N_EDGES = 320000
NC = 2
NS = 16
NW = NC * NS
CHUNK = 128
NCHUNK = N_EDGES // CHUNK
ROWS_PER_SUB = N_NODES // NS
ZR = 125

_mesh = functools.partial(
    plsc.VectorSubcoreMesh, core_axis_name="c", subcore_axis_name="s"
)


def _wid():
    return lax.axis_index("s") * NC + lax.axis_index("c")


@jax.jit
def _sc_degree(dst):
    @functools.partial(
        pl.kernel,
        mesh=_mesh(),
        out_type=jax.ShapeDtypeStruct((NC, N_NODES, 16), jnp.float32),
        scratch_types=[
            pltpu.VMEM((CHUNK,), jnp.int32),
            pltpu.VMEM((CHUNK, 16), jnp.float32),
            pltpu.VMEM_SHARED((N_NODES, 16), jnp.float32),
        ],
    )
    def k(dst_hbm, out_hbm, didx, ones_v, acc_sh):
        cid = lax.axis_index("c")
        sid = lax.axis_index("s")
        wid = _wid()

        @pl.loop(0, CHUNK)
        def _(r):
            ones_v[r, :] = jnp.zeros((16,), jnp.float32)

        @pl.loop(0, 5)
        def _(j):
            pltpu.sync_copy(
                ones_v.at[pl.ds(0, ZR)],
                acc_sh.at[pl.ds(sid * ROWS_PER_SUB + j * ZR, ZR)],
            )

        @pl.loop(0, CHUNK)
        def _(r):
            ones_v[r, :] = jnp.full((16,), 1.0, jnp.float32)

        plsc.subcore_barrier()

        @pl.loop(0, (NCHUNK + NW - 1) // NW)
        def _(g):
            chunk = g * NW + wid

            @pl.when(chunk < NCHUNK)
            def _():
                pltpu.sync_copy(dst_hbm.at[pl.ds(chunk * CHUNK, CHUNK)], didx)
                pltpu.sync_copy(ones_v, acc_sh.at[didx], add=True)

        plsc.subcore_barrier()
        pltpu.sync_copy(
            acc_sh.at[pl.ds(sid * ROWS_PER_SUB, ROWS_PER_SUB)],
            out_hbm.at[cid, pl.ds(sid * ROWS_PER_SUB, ROWS_PER_SUB)],
        )

    return k(dst)


@jax.jit
def _sc_aggregate(hp, src, dst):
    @functools.partial(
        pl.kernel,
        mesh=_mesh(),
        out_type=jax.ShapeDtypeStruct((NC, N_NODES, D), jnp.float32),
        scratch_types=[
            pltpu.VMEM((CHUNK,), jnp.int32),
            pltpu.VMEM((CHUNK,), jnp.int32),
            pltpu.VMEM((CHUNK, D), jnp.float32),
            pltpu.VMEM_SHARED((N_NODES, D), jnp.float32),
            pltpu.SemaphoreType.DMA,
        ],
    )
    def k(hp_hbm, src_hbm, dst_hbm, out_hbm, sidx, didx, rows, acc_sh, sem):
        cid = lax.axis_index("c")
        sid = lax.axis_index("s")
        wid = _wid()

        @pl.loop(0, CHUNK)
        def _(r):
            for cc in range(D // 16):
                rows[r, pl.ds(cc * 16, 16)] = jnp.zeros((16,), jnp.float32)

        @pl.loop(0, 5)
        def _(j):
            pltpu.sync_copy(
                rows.at[pl.ds(0, ZR)],
                acc_sh.at[pl.ds(sid * ROWS_PER_SUB + j * ZR, ZR)],
            )

        plsc.subcore_barrier()

        @pl.loop(0, (NCHUNK + NW - 1) // NW)
        def _(g):
            chunk = g * NW + wid

            @pl.when(chunk < NCHUNK)
            def _():
                base = chunk * CHUNK
                pltpu.sync_copy(src_hbm.at[pl.ds(base, CHUNK)], sidx)
                pltpu.sync_copy(dst_hbm.at[pl.ds(base, CHUNK)], didx)
                pltpu.async_copy(hp_hbm.at[sidx], rows, sem).wait()
                pltpu.sync_copy(rows, acc_sh.at[didx], add=True)

        plsc.subcore_barrier()
        pltpu.sync_copy(
            acc_sh.at[pl.ds(sid * ROWS_PER_SUB, ROWS_PER_SUB)],
            out_hbm.at[cid, pl.ds(sid * ROWS_PER_SUB, ROWS_PER_SUB)],
        )

    return k(hp, src, dst)


def _tc1(x, W1, degp):
    def body(x_ref, w_ref, dp_ref, hp_ref, dinv_ref):
        deg = dp_ref[0, :, 0:1] + dp_ref[1, :, 0:1] + 1.0
        dinv = lax.rsqrt(deg)
        h = jnp.dot(x_ref[...], w_ref[...], preferred_element_type=jnp.float32)
        hp_ref[...] = h * dinv
        dinv_ref[...] = dinv

    return pl.pallas_call(
        body,
        out_shape=(
            jax.ShapeDtypeStruct((N_NODES, D), jnp.float32),
            jax.ShapeDtypeStruct((N_NODES, 1), jnp.float32),
        ),
    )(x, W1, degp)


def _tc2(acc, hp, dinv, W2, b1):
    def body(a_ref, hp_ref, dinv_ref, w_ref, b_ref, out_ref):
        dinv = dinv_ref[...]
        t = (a_ref[0] + a_ref[1] + hp_ref[...]) * dinv + b_ref[...]
        h1 = jnp.maximum(t, 0.0)
        out_ref[...] = (
            jnp.dot(h1, w_ref[...], preferred_element_type=jnp.float32) * dinv
        )

    return pl.pallas_call(
        body,
        out_shape=jax.ShapeDtypeStruct((N_NODES, D), jnp.float32),
    )(acc, hp, dinv, W2, b1)


def _tc3(acc, hp, dinv, b2):
    def body(a_ref, hp_ref, dinv_ref, b_ref, out_ref):
        out_ref[...] = (a_ref[0] + a_ref[1] + hp_ref[...]) * dinv_ref[
            ...
        ] + b_ref[...]

    return pl.pallas_call(
        body,
        out_shape=jax.ShapeDtypeStruct((N_NODES, D), jnp.float32),
    )(acc, hp, dinv, b2)


def kernel(x, edge_index, W1, b1, W2, b2):
    src = edge_index[0].astype(jnp.int32)
    dst = edge_index[1].astype(jnp.int32)
    b1r = b1.reshape(1, D)
    b2r = b2.reshape(1, D)

    degp = _sc_degree(dst)
    h1p, dinv = _tc1(x, W1, degp)
    acc1 = _sc_aggregate(h1p, src, dst)
    h2p = _tc2(acc1, h1p, dinv, W2, b1r)
    acc2 = _sc_aggregate(h2p, src, dst)
    return _tc3(acc2, h2p, dinv, b2r)


# speedup vs baseline: 15.8193x; 15.8193x over previous
import functools

import jax
import jax.numpy as jnp
from jax import lax
from jax.experimental import pallas as pl
from jax.experimental.pallas import tpu as pltpu
from jax.experimental.pallas import tpu_sc as plsc

N_NODES = 10000
D = 128
N_EDGES = 320000
NC = 2
NS = 16
NW = NC * NS
CHUNK = 128
NCHUNK = N_EDGES // CHUNK
N_PAD = 10240
ROWS_PER_SUB = N_PAD // NS
ZR = 128

_mesh = functools.partial(
    plsc.VectorSubcoreMesh, core_axis_name="c", subcore_axis_name="s"
)


def _wid():
    return lax.axis_index("s") * NC + lax.axis_index("c")


@jax.jit
def _sc_degree(dst):
    @functools.partial(
        pl.kernel,
        mesh=_mesh(),
        out_type=jax.ShapeDtypeStruct((NC, N_PAD, 16), jnp.float32),
        scratch_types=[
            pltpu.VMEM((CHUNK,), jnp.int32),
            pltpu.VMEM((CHUNK, 16), jnp.float32),
            pltpu.VMEM_SHARED((N_PAD, 16), jnp.float32),
        ],
    )
    def k(dst_hbm, out_hbm, didx, ones_v, acc_sh):
        cid = lax.axis_index("c")
        sid = lax.axis_index("s")
        wid = _wid()

        @pl.loop(0, CHUNK)
        def _(r):
            ones_v[r, :] = jnp.zeros((16,), jnp.float32)

        @pl.loop(0, 5)
        def _(j):
            pltpu.sync_copy(
                ones_v.at[pl.ds(0, ZR)],
                acc_sh.at[pl.ds(sid * ROWS_PER_SUB + j * ZR, ZR)],
            )

        @pl.loop(0, CHUNK)
        def _(r):
            ones_v[r, :] = jnp.full((16,), 1.0, jnp.float32)

        plsc.subcore_barrier()

        @pl.loop(0, (NCHUNK + NW - 1) // NW)
        def _(g):
            chunk = g * NW + wid

            @pl.when(chunk < NCHUNK)
            def _():
                pltpu.sync_copy(dst_hbm.at[pl.ds(chunk * CHUNK, CHUNK)], didx)
                pltpu.sync_copy(ones_v, acc_sh.at[didx], add=True)

        plsc.subcore_barrier()
        pltpu.sync_copy(
            acc_sh.at[pl.ds(sid * ROWS_PER_SUB, ROWS_PER_SUB)],
            out_hbm.at[cid, pl.ds(sid * ROWS_PER_SUB, ROWS_PER_SUB)],
        )

    return k(dst)


@jax.jit
def _sc_aggregate(hp, src, dst):
    @functools.partial(
        pl.kernel,
        mesh=_mesh(),
        out_type=jax.ShapeDtypeStruct((NC, N_PAD, D), jnp.float32),
        scratch_types=[
            pltpu.VMEM((CHUNK,), jnp.int32),
            pltpu.VMEM((CHUNK,), jnp.int32),
            pltpu.VMEM((CHUNK, D), jnp.float32),
            pltpu.VMEM_SHARED((N_PAD, D), jnp.float32),
            pltpu.SemaphoreType.DMA,
        ],
    )
    def k(hp_hbm, src_hbm, dst_hbm, out_hbm, sidx, didx, rows, acc_sh, sem):
        cid = lax.axis_index("c")
        sid = lax.axis_index("s")
        wid = _wid()

        @pl.loop(0, CHUNK)
        def _(r):
            for cc in range(D // 16):
                rows[r, pl.ds(cc * 16, 16)] = jnp.zeros((16,), jnp.float32)

        @pl.loop(0, 5)
        def _(j):
            pltpu.sync_copy(
                rows.at[pl.ds(0, ZR)],
                acc_sh.at[pl.ds(sid * ROWS_PER_SUB + j * ZR, ZR)],
            )

        plsc.subcore_barrier()

        @pl.loop(0, (NCHUNK + NW - 1) // NW)
        def _(g):
            chunk = g * NW + wid

            @pl.when(chunk < NCHUNK)
            def _():
                base = chunk * CHUNK
                pltpu.sync_copy(src_hbm.at[pl.ds(base, CHUNK)], sidx)
                pltpu.sync_copy(dst_hbm.at[pl.ds(base, CHUNK)], didx)
                pltpu.async_copy(hp_hbm.at[sidx], rows, sem).wait()
                pltpu.sync_copy(rows, acc_sh.at[didx], add=True)

        plsc.subcore_barrier()
        pltpu.sync_copy(
            acc_sh.at[pl.ds(sid * ROWS_PER_SUB, ROWS_PER_SUB)],
            out_hbm.at[cid, pl.ds(sid * ROWS_PER_SUB, ROWS_PER_SUB)],
        )

    return k(hp, src, dst)


def _tc1(x, W1, degp):
    def body(x_ref, w_ref, dp_ref, hp_ref, dinv_ref):
        deg = dp_ref[0, :, 0:1] + dp_ref[1, :, 0:1] + 1.0
        dinv = lax.rsqrt(deg)
        h = jnp.dot(x_ref[...], w_ref[...], preferred_element_type=jnp.float32)
        hp_ref[...] = h * dinv
        dinv_ref[...] = dinv

    return pl.pallas_call(
        body,
        out_shape=(
            jax.ShapeDtypeStruct((N_NODES, D), jnp.float32),
            jax.ShapeDtypeStruct((N_NODES, 1), jnp.float32),
        ),
    )(x, W1, degp)


def _tc2(acc, hp, dinv, W2, b1):
    def body(a_ref, hp_ref, dinv_ref, w_ref, b_ref, out_ref):
        dinv = dinv_ref[...]
        t = (a_ref[0] + a_ref[1] + hp_ref[...]) * dinv + b_ref[...]
        h1 = jnp.maximum(t, 0.0)
        out_ref[...] = (
            jnp.dot(h1, w_ref[...], preferred_element_type=jnp.float32) * dinv
        )

    return pl.pallas_call(
        body,
        out_shape=jax.ShapeDtypeStruct((N_NODES, D), jnp.float32),
    )(acc, hp, dinv, W2, b1)


def _tc3(acc, hp, dinv, b2):
    def body(a_ref, hp_ref, dinv_ref, b_ref, out_ref):
        out_ref[...] = (a_ref[0] + a_ref[1] + hp_ref[...]) * dinv_ref[
            ...
        ] + b_ref[...]

    return pl.pallas_call(
        body,
        out_shape=jax.ShapeDtypeStruct((N_NODES, D), jnp.float32),
    )(acc, hp, dinv, b2)


def kernel(x, edge_index, W1, b1, W2, b2):
    src = edge_index[0].astype(jnp.int32)
    dst = edge_index[1].astype(jnp.int32)
    b1r = b1.reshape(1, D)
    b2r = b2.reshape(1, D)

    degp = _sc_degree(dst)[:, :N_NODES]
    h1p, dinv = _tc1(x, W1, degp)
    acc1 = _sc_aggregate(h1p, src, dst)[:, :N_NODES]
    h2p = _tc2(acc1, h1p, dinv, W2, b1r)
    acc2 = _sc_aggregate(h2p, src, dst)[:, :N_NODES]
    return _tc3(acc2, h2p, dinv, b2r)
